# baseline (speedup 1.0000x reference)
import jax
import jax.numpy as jnp
from jax.experimental import pallas as pl
from jax.experimental.pallas import tpu as pltpu

_TOPK = 2
_CHUNK = 1024
_NBUF = 3


def _stream_kernel(x_hbm, o_hbm, in_buf, out_buf, in_sems, out_sems):
    n_chunks = x_hbm.shape[0] // _CHUNK

    def in_cp(i, s):
        return pltpu.make_async_copy(
            x_hbm.at[pl.ds(i * _CHUNK, _CHUNK), :], in_buf.at[s], in_sems.at[s])

    def out_cp(i, s):
        return pltpu.make_async_copy(
            out_buf.at[s], o_hbm.at[pl.ds(i * _CHUNK, _CHUNK), :], out_sems.at[s])

    for s in range(_NBUF):
        in_cp(s, s).start()
    for i in range(n_chunks):
        s = i % _NBUF
        in_cp(i, s).wait()
        if i >= _NBUF:
            out_cp(i - _NBUF, s).wait()
        out_buf[s] = in_buf[s] * jnp.float32(_TOPK)
        out_cp(i, s).start()
        if i + _NBUF < n_chunks:
            in_cp(i + _NBUF, s).start()
    for i in range(n_chunks - _NBUF, n_chunks):
        out_cp(i, i % _NBUF).wait()


@jax.jit
def kernel(input, exp_indices):
    T, H = input.shape
    del exp_indices
    return pl.pallas_call(
        _stream_kernel,
        in_specs=[pl.BlockSpec(memory_space=pltpu.MemorySpace.HBM)],
        out_specs=pl.BlockSpec(memory_space=pltpu.MemorySpace.HBM),
        out_shape=jax.ShapeDtypeStruct((T, H), input.dtype),
        scratch_shapes=[
            pltpu.VMEM((_NBUF, _CHUNK, H), jnp.float32),
            pltpu.VMEM((_NBUF, _CHUNK, H), jnp.float32),
            pltpu.SemaphoreType.DMA((_NBUF,)),
            pltpu.SemaphoreType.DMA((_NBUF,)),
        ],
    )(input)


# speedup vs baseline: 1.0062x; 1.0062x over previous
import jax
import jax.numpy as jnp
from jax.experimental import pallas as pl
from jax.experimental.pallas import tpu as pltpu

_TOPK = 2
_CHUNK = 2048
_NBUF = 3


def _stream_kernel(x_hbm, o_hbm, in_buf, out_buf, in_sems, out_sems):
    n_chunks = x_hbm.shape[0] // _CHUNK

    def in_cp(i, s):
        return pltpu.make_async_copy(
            x_hbm.at[pl.ds(i * _CHUNK, _CHUNK), :], in_buf.at[s], in_sems.at[s])

    def out_cp(i, s):
        return pltpu.make_async_copy(
            out_buf.at[s], o_hbm.at[pl.ds(i * _CHUNK, _CHUNK), :], out_sems.at[s])

    for s in range(_NBUF):
        in_cp(s, s).start()
    for i in range(n_chunks):
        s = i % _NBUF
        in_cp(i, s).wait()
        if i >= _NBUF:
            out_cp(i - _NBUF, s).wait()
        out_buf[s] = in_buf[s] * jnp.float32(_TOPK)
        out_cp(i, s).start()
        if i + _NBUF < n_chunks:
            in_cp(i + _NBUF, s).start()
    for i in range(n_chunks - _NBUF, n_chunks):
        out_cp(i, i % _NBUF).wait()


@jax.jit
def kernel(input, exp_indices):
    T, H = input.shape
    del exp_indices
    return pl.pallas_call(
        _stream_kernel,
        in_specs=[pl.BlockSpec(memory_space=pltpu.MemorySpace.HBM)],
        out_specs=pl.BlockSpec(memory_space=pltpu.MemorySpace.HBM),
        out_shape=jax.ShapeDtypeStruct((T, H), input.dtype),
        scratch_shapes=[
            pltpu.VMEM((_NBUF, _CHUNK, H), jnp.float32),
            pltpu.VMEM((_NBUF, _CHUNK, H), jnp.float32),
            pltpu.SemaphoreType.DMA((_NBUF,)),
            pltpu.SemaphoreType.DMA((_NBUF,)),
        ],
    )(input)


# speedup vs baseline: 1.0249x; 1.0186x over previous
import jax
import jax.numpy as jnp
from jax.experimental import pallas as pl

_TOPK = 2
_BLOCK_ROWS = 3584


def _scale_kernel(x_ref, o_ref):
    o_ref[...] = x_ref[...] * jnp.float32(_TOPK)


@jax.jit
def kernel(input, exp_indices):
    T, H = input.shape
    del exp_indices
    grid = (pl.cdiv(T, _BLOCK_ROWS),)
    return pl.pallas_call(
        _scale_kernel,
        grid=grid,
        in_specs=[pl.BlockSpec((_BLOCK_ROWS, H), lambda i: (i, 0))],
        out_specs=pl.BlockSpec((_BLOCK_ROWS, H), lambda i: (i, 0)),
        out_shape=jax.ShapeDtypeStruct((T, H), input.dtype),
    )(input)
